# Initial kernel scaffold; baseline (speedup 1.0000x reference)
import jax
import jax.numpy as jnp
from jax.experimental import pallas as pl


def kernel(x_local, x_global, edge_index, batch, W_local, b_local, W_global, b_global, W_mix, b_mix, W_msg, b_msg, W_self, b_self, W_out, b_out):
    raise NotImplementedError("write your pallas kernel here")


# speedup vs baseline: 4.7670x; 4.7670x over previous
import functools

import jax
import jax.numpy as jnp
from jax import lax
from jax.experimental import pallas as pl
from jax.experimental.pallas import tpu as pltpu
from jax.experimental.pallas import tpu_sc as plsc

N = 100000
E = 1600000
B = 128
HIDDEN = 32
HALF = 16
NUM_CLASSES = 2

ROWS = 512
GRID1 = 196
NPAD = GRID1 * ROWS

NS = 16
ROWS_PER_TILE = NPAD // NS
CHUNK = 128
EPT_CHUNKS = 782
EDGES_PER_TILE = EPT_CHUNKS * CHUNK
EPAD = NS * EDGES_PER_TILE


def _enc_body(xb_ref, bt_ref, xg_ref, wl_ref, bl_ref, wg_ref, bg_ref,
              wm_ref, bm_ref, wmsg_ref, bmsg_ref, wself_ref, bself_ref,
              m0a_ref, m0b_ref, t0a_ref, t0b_ref):
    f32 = jnp.float32
    g = jnp.maximum(
        jnp.dot(xg_ref[...], wg_ref[...], preferred_element_type=f32) + bg_ref[...], 0.0)
    hl = jnp.maximum(
        jnp.dot(xb_ref[...], wl_ref[...], preferred_element_type=f32) + bl_ref[...], 0.0)
    bt = bt_ref[0, 0, :]
    oh = (bt[:, None] == lax.broadcasted_iota(jnp.int32, (ROWS, B), 1)).astype(f32)
    hg = jnp.dot(oh, g, preferred_element_type=f32)
    hcat = jnp.concatenate([hl, hg, hl * hg], axis=1)
    h0 = jnp.maximum(
        jnp.dot(hcat, wm_ref[...], preferred_element_type=f32) + bm_ref[...], 0.0)
    m0 = jnp.maximum(
        jnp.dot(h0, wmsg_ref[...], preferred_element_type=f32) + bmsg_ref[...], 0.0)
    t0 = m0 + jnp.dot(h0, wself_ref[...], preferred_element_type=f32) + bself_ref[...]
    m0a_ref[...] = m0[:, :HALF]
    m0b_ref[...] = m0[:, HALF:]
    t0a_ref[...] = t0[:, :HALF]
    t0b_ref[...] = t0[:, HALF:]


def _whole(shape):
    nd = len(shape)
    return pl.BlockSpec(shape, lambda i: (0,) * nd)


def _encoder(x_local, batch3, x_global, wl, bl, wg, bg, wm, bm, wmsg, bmsg, wself, bself):
    out = jax.ShapeDtypeStruct((NPAD, HALF), jnp.float32)
    return pl.pallas_call(
        _enc_body,
        grid=(GRID1,),
        in_specs=[
            pl.BlockSpec((ROWS, 128), lambda i: (i, 0)),
            pl.BlockSpec((1, 1, ROWS), lambda i: (i, 0, 0)),
            _whole((B, 64)),
            _whole((128, HIDDEN)), _whole((1, HIDDEN)),
            _whole((64, HIDDEN)), _whole((1, HIDDEN)),
            _whole((3 * HIDDEN, HIDDEN)), _whole((1, HIDDEN)),
            _whole((HIDDEN, HIDDEN)), _whole((1, HIDDEN)),
            _whole((HIDDEN, HIDDEN)), _whole((1, HIDDEN)),
        ],
        out_specs=[pl.BlockSpec((ROWS, HALF), lambda i: (i, 0))] * 4,
        out_shape=[out, out, out, out],
    )(x_local, batch3, x_global, wl, bl, wg, bg, wm, bm, wmsg, bmsg, wself, bself)


def _sc_body(m0a, m0b, t0a, t0b, srce, dste, outa, outb, sidx, didx, rows, acc, sem):
    c = lax.axis_index("c")
    s = lax.axis_index("s")

    def run(m0, t0, out):
        r0 = s * ROWS_PER_TILE
        pltpu.sync_copy(t0.at[pl.ds(r0, ROWS_PER_TILE)],
                        acc.at[pl.ds(r0, ROWS_PER_TILE)])
        plsc.subcore_barrier()
        e0 = s * EDGES_PER_TILE

        def body(j, carry):
            base = e0 + j * CHUNK
            pltpu.sync_copy(srce.at[pl.ds(base, CHUNK)], sidx.at[0])
            pltpu.sync_copy(dste.at[pl.ds(base, CHUNK)], didx.at[0])
            pltpu.async_copy(m0.at[sidx.at[0]], rows, sem).wait()
            pltpu.sync_copy(rows, acc.at[didx.at[0]], add=True)
            return carry

        lax.fori_loop(0, EPT_CHUNKS, body, 0)
        plsc.subcore_barrier()
        pltpu.sync_copy(acc.at[pl.ds(r0, ROWS_PER_TILE)],
                        out.at[pl.ds(r0, ROWS_PER_TILE)])

    @pl.when(c == 0)
    def _():
        run(m0a, t0a, outa)

    @pl.when(c == 1)
    def _():
        run(m0b, t0b, outb)


@functools.partial(
    pl.kernel,
    out_type=(jax.ShapeDtypeStruct((NPAD, HALF), jnp.float32),) * 2,
    mesh=plsc.VectorSubcoreMesh(core_axis_name="c", subcore_axis_name="s"),
    scratch_types=[
        pltpu.VMEM((1, CHUNK), jnp.int32),
        pltpu.VMEM((1, CHUNK), jnp.int32),
        pltpu.VMEM((CHUNK, HALF), jnp.float32),
        pltpu.VMEM_SHARED((NPAD, HALF), jnp.float32),
        pltpu.SemaphoreType.DMA,
    ],
    compiler_params=pltpu.CompilerParams(use_tc_tiling_on_sc=False),
)
def _sc_edge(m0a, m0b, t0a, t0b, srce, dste, outa, outb, sidx, didx, rows, acc, sem):
    _sc_body(m0a, m0b, t0a, t0b, srce, dste, outa, outb, sidx, didx, rows, acc, sem)


def _head_body(a_ref, b_ref, w_ref, bo_ref, out_ref):
    h = jnp.maximum(jnp.concatenate([a_ref[...], b_ref[...]], axis=1), 0.0)
    out_ref[...] = (
        jnp.dot(h, w_ref[...], preferred_element_type=jnp.float32) + bo_ref[...])


def _head(outa, outb, w_out, b_out):
    return pl.pallas_call(
        _head_body,
        grid=(GRID1,),
        in_specs=[
            pl.BlockSpec((ROWS, HALF), lambda i: (i, 0)),
            pl.BlockSpec((ROWS, HALF), lambda i: (i, 0)),
            _whole((HIDDEN, NUM_CLASSES)),
            _whole((1, NUM_CLASSES)),
        ],
        out_specs=pl.BlockSpec((ROWS, NUM_CLASSES), lambda i: (i, 0)),
        out_shape=jax.ShapeDtypeStruct((N, NUM_CLASSES), jnp.float32),
    )(outa, outb, w_out, b_out)


@jax.jit
def kernel(x_local, x_global, edge_index, batch, W_local, b_local, W_global,
           b_global, W_mix, b_mix, W_msg, b_msg, W_self, b_self, W_out, b_out):
    batch3 = jnp.pad(batch, (0, NPAD - N)).reshape(GRID1, 1, ROWS)
    pad_e = jnp.full((EPAD - E,), N, jnp.int32)
    srce = jnp.concatenate([edge_index[0], pad_e])
    dste = jnp.concatenate([edge_index[1], pad_e])
    row = lambda v: v.reshape(1, -1)

    m0a, m0b, t0a, t0b = _encoder(
        x_local, batch3, x_global, W_local, row(b_local), W_global, row(b_global),
        W_mix, row(b_mix), W_msg, row(b_msg), W_self, row(b_self))
    outa, outb = _sc_edge(m0a, m0b, t0a, t0b, srce, dste)
    return _head(outa, outb, W_out, row(b_out))


# speedup vs baseline: 10.1486x; 2.1289x over previous
import functools

import jax
import jax.numpy as jnp
from jax import lax
from jax.experimental import pallas as pl
from jax.experimental.pallas import tpu as pltpu
from jax.experimental.pallas import tpu_sc as plsc

N = 100000
E = 1600000
B = 128
HIDDEN = 32
HALF = 16
NUM_CLASSES = 2

ROWS = 512
GRID1 = 196
NPAD = GRID1 * ROWS

NS = 16
ROWS_PER_TILE = NPAD // NS
CHUNK = 128
IB = 4
NBLK = 196
CHUNKS_PER_TILE = IB * NBLK
EDGES_PER_TILE = CHUNKS_PER_TILE * CHUNK
EPAD = NS * EDGES_PER_TILE


def _enc_body(xb_ref, bt_ref, xg_ref, wl_ref, bl_ref, wg_ref, bg_ref,
              wm_ref, bm_ref, wmsg_ref, bmsg_ref, wself_ref, bself_ref,
              m0a_ref, m0b_ref, t0a_ref, t0b_ref):
    f32 = jnp.float32
    g = jnp.maximum(
        jnp.dot(xg_ref[...], wg_ref[...], preferred_element_type=f32) + bg_ref[...], 0.0)
    hl = jnp.maximum(
        jnp.dot(xb_ref[...], wl_ref[...], preferred_element_type=f32) + bl_ref[...], 0.0)
    bt = bt_ref[0, 0, :]
    oh = (bt[:, None] == lax.broadcasted_iota(jnp.int32, (ROWS, B), 1)).astype(f32)
    hg = jnp.dot(oh, g, preferred_element_type=f32)
    hcat = jnp.concatenate([hl, hg, hl * hg], axis=1)
    h0 = jnp.maximum(
        jnp.dot(hcat, wm_ref[...], preferred_element_type=f32) + bm_ref[...], 0.0)
    m0 = jnp.maximum(
        jnp.dot(h0, wmsg_ref[...], preferred_element_type=f32) + bmsg_ref[...], 0.0)
    t0 = m0 + jnp.dot(h0, wself_ref[...], preferred_element_type=f32) + bself_ref[...]
    m0a_ref[...] = m0[:, :HALF]
    m0b_ref[...] = m0[:, HALF:]
    t0a_ref[...] = t0[:, :HALF]
    t0b_ref[...] = t0[:, HALF:]


def _whole(shape):
    nd = len(shape)
    return pl.BlockSpec(shape, lambda i: (0,) * nd)


def _encoder(x_local, batch3, x_global, wl, bl, wg, bg, wm, bm, wmsg, bmsg, wself, bself):
    out = jax.ShapeDtypeStruct((NPAD, HALF), jnp.float32)
    return pl.pallas_call(
        _enc_body,
        grid=(GRID1,),
        in_specs=[
            pl.BlockSpec((ROWS, 128), lambda i: (i, 0)),
            pl.BlockSpec((1, 1, ROWS), lambda i: (i, 0, 0)),
            _whole((B, 64)),
            _whole((128, HIDDEN)), _whole((1, HIDDEN)),
            _whole((64, HIDDEN)), _whole((1, HIDDEN)),
            _whole((3 * HIDDEN, HIDDEN)), _whole((1, HIDDEN)),
            _whole((HIDDEN, HIDDEN)), _whole((1, HIDDEN)),
            _whole((HIDDEN, HIDDEN)), _whole((1, HIDDEN)),
        ],
        out_specs=[pl.BlockSpec((ROWS, HALF), lambda i: (i, 0))] * 4,
        out_shape=[out, out, out, out],
    )(x_local, batch3, x_global, wl, bl, wg, bg, wm, bm, wmsg, bmsg, wself, bself)


def _sc_body(m0a, m0b, t0a, t0b, src2d, dst2d, outa, outb,
             sidx, didx, rows, acc, semg0, semg1, sems0, sems1):
    c = lax.axis_index("c")
    s = lax.axis_index("s")
    semg = (semg0, semg1)
    sems = (sems0, sems1)

    def run(m0, t0, out):
        r0 = s * ROWS_PER_TILE
        pltpu.sync_copy(t0.at[pl.ds(r0, ROWS_PER_TILE)],
                        acc.at[pl.ds(r0, ROWS_PER_TILE)])
        plsc.subcore_barrier()
        row_base = s * CHUNKS_PER_TILE

        def load_idx(b, p):
            r = row_base + b * IB
            pltpu.sync_copy(src2d.at[pl.ds(r, IB)], sidx.at[p])
            pltpu.sync_copy(dst2d.at[pl.ds(r, IB)], didx.at[p])

        def fire_gathers(p):
            for jj in range(IB):
                pltpu.async_copy(m0.at[sidx.at[p, jj]], rows.at[p, jj], semg[p])

        def drain_gathers(p):
            for jj in range(IB):
                pltpu.make_async_copy(
                    m0.at[sidx.at[p, jj]], rows.at[p, jj], semg[p]).wait()

        def fire_scatters(p):
            for jj in range(IB):
                pltpu.async_copy(rows.at[p, jj], acc.at[didx.at[p, jj]],
                                 sems[p], add=True)

        def drain_scatters(p):
            for jj in range(IB):
                pltpu.make_async_copy(
                    rows.at[p, jj], acc.at[didx.at[p, jj]], sems[p]).wait()

        load_idx(jnp.int32(0), 0)
        fire_gathers(0)

        def pair2(bb, carry):
            b0 = 2 * bb
            @pl.when(bb > 0)
            def _():
                drain_scatters(1)
            load_idx(b0 + 1, 1)
            fire_gathers(1)
            drain_gathers(0)
            fire_scatters(0)
            drain_scatters(0)
            load_idx(b0 + 2, 0)
            fire_gathers(0)
            drain_gathers(1)
            fire_scatters(1)
            return carry

        npair = NBLK // 2 - 1
        lax.fori_loop(0, npair, pair2, 0)
        drain_scatters(1)
        load_idx(jnp.int32(NBLK - 1), 1)
        fire_gathers(1)
        drain_gathers(0)
        fire_scatters(0)
        drain_scatters(0)
        drain_gathers(1)
        fire_scatters(1)
        drain_scatters(1)
        plsc.subcore_barrier()
        pltpu.sync_copy(acc.at[pl.ds(r0, ROWS_PER_TILE)],
                        out.at[pl.ds(r0, ROWS_PER_TILE)])

    @pl.when(c == 0)
    def _():
        run(m0a, t0a, outa)

    @pl.when(c == 1)
    def _():
        run(m0b, t0b, outb)


@functools.partial(
    pl.kernel,
    out_type=(jax.ShapeDtypeStruct((NPAD, HALF), jnp.float32),) * 2,
    mesh=plsc.VectorSubcoreMesh(core_axis_name="c", subcore_axis_name="s"),
    scratch_types=[
        pltpu.VMEM((2, IB, CHUNK), jnp.int32),
        pltpu.VMEM((2, IB, CHUNK), jnp.int32),
        pltpu.VMEM((2, IB, CHUNK, HALF), jnp.float32),
        pltpu.VMEM_SHARED((NPAD, HALF), jnp.float32),
        pltpu.SemaphoreType.DMA,
        pltpu.SemaphoreType.DMA,
        pltpu.SemaphoreType.DMA,
        pltpu.SemaphoreType.DMA,
    ],
    compiler_params=pltpu.CompilerParams(use_tc_tiling_on_sc=False),
)
def _sc_edge(m0a, m0b, t0a, t0b, src2d, dst2d, outa, outb,
             sidx, didx, rows, acc, semg0, semg1, sems0, sems1):
    _sc_body(m0a, m0b, t0a, t0b, src2d, dst2d, outa, outb,
             sidx, didx, rows, acc, semg0, semg1, sems0, sems1)


def _head_body(a_ref, b_ref, w_ref, bo_ref, out_ref):
    h = jnp.maximum(jnp.concatenate([a_ref[...], b_ref[...]], axis=1), 0.0)
    out_ref[...] = (
        jnp.dot(h, w_ref[...], preferred_element_type=jnp.float32) + bo_ref[...])


def _head(outa, outb, w_out, b_out):
    return pl.pallas_call(
        _head_body,
        grid=(GRID1,),
        in_specs=[
            pl.BlockSpec((ROWS, HALF), lambda i: (i, 0)),
            pl.BlockSpec((ROWS, HALF), lambda i: (i, 0)),
            _whole((HIDDEN, NUM_CLASSES)),
            _whole((1, NUM_CLASSES)),
        ],
        out_specs=pl.BlockSpec((ROWS, NUM_CLASSES), lambda i: (i, 0)),
        out_shape=jax.ShapeDtypeStruct((N, NUM_CLASSES), jnp.float32),
    )(outa, outb, w_out, b_out)


@jax.jit
def kernel(x_local, x_global, edge_index, batch, W_local, b_local, W_global,
           b_global, W_mix, b_mix, W_msg, b_msg, W_self, b_self, W_out, b_out):
    batch3 = jnp.pad(batch, (0, NPAD - N)).reshape(GRID1, 1, ROWS)
    pad_e = jnp.full((EPAD - E,), N, jnp.int32)
    srce = jnp.concatenate([edge_index[0], pad_e]).reshape(EPAD // CHUNK, CHUNK)
    dste = jnp.concatenate([edge_index[1], pad_e]).reshape(EPAD // CHUNK, CHUNK)
    row = lambda v: v.reshape(1, -1)

    m0a, m0b, t0a, t0b = _encoder(
        x_local, batch3, x_global, W_local, row(b_local), W_global, row(b_global),
        W_mix, row(b_mix), W_msg, row(b_msg), W_self, row(b_self))
    outa, outb = _sc_edge(m0a, m0b, t0a, t0b, srce, dste)
    return _head(outa, outb, W_out, row(b_out))


# speedup vs baseline: 10.5152x; 1.0361x over previous
import functools

import jax
import jax.numpy as jnp
from jax import lax
from jax.experimental import pallas as pl
from jax.experimental.pallas import tpu as pltpu
from jax.experimental.pallas import tpu_sc as plsc

N = 100000
E = 1600000
B = 128
HIDDEN = 32
HALF = 16
NUM_CLASSES = 2

ROWS = 512
GRID1 = 196
NPAD = GRID1 * ROWS

NS = 16
ROWS_PER_TILE = NPAD // NS
CHUNK = 128
IB = 4
NBLK = 196
CHUNKS_PER_TILE = IB * NBLK
EDGES_PER_TILE = CHUNKS_PER_TILE * CHUNK
EPAD = NS * EDGES_PER_TILE


def _enc_body(xb_ref, bt_ref, xgt_ref, wl_ref, bl_ref, wgt_ref, bg_ref,
              wmt_ref, bm_ref, wmsgt_ref, bmsg_ref, wselft_ref, bself_ref,
              m0at_ref, m0bt_ref, t0t_ref):
    f32 = jnp.float32
    gt = jnp.maximum(
        jnp.dot(wgt_ref[...], xgt_ref[...], preferred_element_type=f32)
        + bg_ref[...], 0.0)
    hl = jnp.maximum(
        jnp.dot(xb_ref[...], wl_ref[...], preferred_element_type=f32)
        + bl_ref[...], 0.0)
    hlt = hl.T
    bt = bt_ref[0, 0, :]
    oht = (bt[None, :] == lax.broadcasted_iota(jnp.int32, (B, ROWS), 0)).astype(f32)
    hgt = jnp.dot(gt, oht, preferred_element_type=f32)
    hcatt = jnp.concatenate([hlt, hgt, hlt * hgt], axis=0)
    h0t = jnp.maximum(
        jnp.dot(wmt_ref[...], hcatt, preferred_element_type=f32)
        + bm_ref[...], 0.0)
    m0t = jnp.maximum(
        jnp.dot(wmsgt_ref[...], h0t, preferred_element_type=f32)
        + bmsg_ref[...], 0.0)
    t0t = m0t + (jnp.dot(wselft_ref[...], h0t, preferred_element_type=f32)
                 + bself_ref[...])
    m0at_ref[...] = m0t[:HALF, :]
    m0bt_ref[...] = m0t[HALF:, :]
    t0t_ref[...] = t0t


def _whole(shape):
    nd = len(shape)
    return pl.BlockSpec(shape, lambda i: (0,) * nd)


def _encoder(x_local, batch3, x_globalt, wl, bl, wgt, bg, wmt, bm,
             wmsgt, bmsg, wselft, bself):
    outh = jax.ShapeDtypeStruct((HALF, NPAD), jnp.float32)
    outf = jax.ShapeDtypeStruct((HIDDEN, NPAD), jnp.float32)
    return pl.pallas_call(
        _enc_body,
        grid=(GRID1,),
        in_specs=[
            pl.BlockSpec((ROWS, 128), lambda i: (i, 0)),
            pl.BlockSpec((1, 1, ROWS), lambda i: (i, 0, 0)),
            _whole((64, B)),
            _whole((128, HIDDEN)), _whole((1, HIDDEN)),
            _whole((HIDDEN, 64)), _whole((HIDDEN, 1)),
            _whole((HIDDEN, 3 * HIDDEN)), _whole((HIDDEN, 1)),
            _whole((HIDDEN, HIDDEN)), _whole((HIDDEN, 1)),
            _whole((HIDDEN, HIDDEN)), _whole((HIDDEN, 1)),
        ],
        out_specs=[pl.BlockSpec((HALF, ROWS), lambda i: (0, i)),
                   pl.BlockSpec((HALF, ROWS), lambda i: (0, i)),
                   pl.BlockSpec((HIDDEN, ROWS), lambda i: (0, i))],
        out_shape=[outh, outh, outf],
    )(x_local, batch3, x_globalt, wl, bl, wgt, bg, wmt, bm, wmsgt, bmsg,
      wselft, bself)


def _sc_body(m0a, m0b, t0, src2d, dst2d, outa, outb,
             sidx, didx, rows, acc, semg0, semg1, sems0, sems1):
    c = lax.axis_index("c")
    s = lax.axis_index("s")
    semg = (semg0, semg1)
    sems = (sems0, sems1)

    def run(m0, col, out):
        r0 = s * ROWS_PER_TILE
        pltpu.sync_copy(t0.at[pl.ds(r0, ROWS_PER_TILE), pl.ds(col, HALF)],
                        acc.at[pl.ds(r0, ROWS_PER_TILE)])
        plsc.subcore_barrier()
        row_base = s * CHUNKS_PER_TILE

        def load_idx(b, p):
            pltpu.sync_copy(src2d.at[pl.ds(row_base + b * IB, IB)], sidx.at[p])
            pltpu.sync_copy(dst2d.at[pl.ds(row_base + b * IB, IB)], didx.at[p])

        def fire_gathers(p):
            for jj in range(IB):
                pltpu.async_copy(
                    m0.at[sidx.at[p, jj]], rows.at[p, jj], semg[p])

        def drain_gathers(p):
            for jj in range(IB):
                pltpu.make_async_copy(
                    m0.at[sidx.at[p, jj]], rows.at[p, jj], semg[p]).wait()

        def fire_scatters(p):
            for jj in range(IB):
                pltpu.async_copy(rows.at[p, jj], acc.at[didx.at[p, jj]],
                                 sems[p], add=True)

        def drain_scatters(p):
            for jj in range(IB):
                pltpu.make_async_copy(
                    rows.at[p, jj], acc.at[didx.at[p, jj]], sems[p]).wait()

        load_idx(jnp.int32(0), 0)
        fire_gathers(0)

        def pair2(bb, carry):
            b0 = 2 * bb
            @pl.when(bb > 0)
            def _():
                drain_scatters(1)
            load_idx(b0 + 1, 1)
            fire_gathers(1)
            drain_gathers(0)
            fire_scatters(0)
            drain_scatters(0)
            load_idx(b0 + 2, 0)
            fire_gathers(0)
            drain_gathers(1)
            fire_scatters(1)
            return carry

        npair = NBLK // 2 - 1
        lax.fori_loop(0, npair, pair2, 0)
        drain_scatters(1)
        load_idx(jnp.int32(NBLK - 1), 1)
        fire_gathers(1)
        drain_gathers(0)
        fire_scatters(0)
        drain_scatters(0)
        drain_gathers(1)
        fire_scatters(1)
        drain_scatters(1)
        plsc.subcore_barrier()
        pltpu.sync_copy(acc.at[pl.ds(r0, ROWS_PER_TILE)],
                        out.at[pl.ds(r0, ROWS_PER_TILE)])

    @pl.when(c == 0)
    def _():
        run(m0a, 0, outa)

    @pl.when(c == 1)
    def _():
        run(m0b, HALF, outb)


@functools.partial(
    pl.kernel,
    out_type=(jax.ShapeDtypeStruct((NPAD, HALF), jnp.float32),) * 2,
    mesh=plsc.VectorSubcoreMesh(core_axis_name="c", subcore_axis_name="s"),
    scratch_types=[
        pltpu.VMEM((2, IB, CHUNK), jnp.int32),
        pltpu.VMEM((2, IB, CHUNK), jnp.int32),
        pltpu.VMEM((2, IB, CHUNK, HALF), jnp.float32),
        pltpu.VMEM_SHARED((NPAD, HALF), jnp.float32),
        pltpu.SemaphoreType.DMA,
        pltpu.SemaphoreType.DMA,
        pltpu.SemaphoreType.DMA,
        pltpu.SemaphoreType.DMA,
    ],
    compiler_params=pltpu.CompilerParams(use_tc_tiling_on_sc=False),
)
def _sc_edge(m0a, m0b, t0, src2d, dst2d, outa, outb,
             sidx, didx, rows, acc, semg0, semg1, sems0, sems1):
    _sc_body(m0a, m0b, t0, src2d, dst2d, outa, outb,
             sidx, didx, rows, acc, semg0, semg1, sems0, sems1)


def _head_body(a_ref, b_ref, wa_ref, wb_ref, bo_ref, out_ref):
    ha = jnp.maximum(a_ref[...], 0.0)
    hb = jnp.maximum(b_ref[...], 0.0)
    out_ref[...] = (jnp.dot(ha, wa_ref[...], preferred_element_type=jnp.float32)
                    + jnp.dot(hb, wb_ref[...], preferred_element_type=jnp.float32)
                    + bo_ref[...])


def _head(pa, pb, wbig_a, wbig_b, bbig):
    return pl.pallas_call(
        _head_body,
        grid=(GRID1,),
        in_specs=[
            pl.BlockSpec((ROWS // 8, 128), lambda i: (i, 0)),
            pl.BlockSpec((ROWS // 8, 128), lambda i: (i, 0)),
            _whole((128, 8 * NUM_CLASSES)),
            _whole((128, 8 * NUM_CLASSES)),
            _whole((1, 8 * NUM_CLASSES)),
        ],
        out_specs=pl.BlockSpec((ROWS // 8, 8 * NUM_CLASSES), lambda i: (i, 0)),
        out_shape=jax.ShapeDtypeStruct((N * NUM_CLASSES // 16, 8 * NUM_CLASSES),
                                       jnp.float32),
    )(pa, pb, wbig_a, wbig_b, bbig)


@jax.jit
def kernel(x_local, x_global, edge_index, batch, W_local, b_local, W_global,
           b_global, W_mix, b_mix, W_msg, b_msg, W_self, b_self, W_out, b_out):
    batch3 = jnp.pad(batch, (0, NPAD - N)).reshape(GRID1, 1, ROWS)
    pad_e = jnp.full((EPAD - E,), N, jnp.int32)
    src2d = jnp.concatenate([edge_index[0], pad_e]).reshape(EPAD // CHUNK, CHUNK)
    dst2d = jnp.concatenate([edge_index[1], pad_e]).reshape(EPAD // CHUNK, CHUNK)
    row = lambda v: v.reshape(1, -1)

    col = lambda v: v.reshape(-1, 1)
    m0at, m0bt, t0t = _encoder(
        x_local, batch3, x_global.T, W_local, row(b_local), W_global.T,
        col(b_global), W_mix.T, col(b_mix), W_msg.T, col(b_msg),
        W_self.T, col(b_self))
    outa, outb = _sc_edge(m0at.T, m0bt.T, t0t.T, src2d, dst2d)

    eye8 = jnp.eye(8, dtype=jnp.float32)
    wbig_a = jnp.kron(eye8, W_out[:HALF])
    wbig_b = jnp.kron(eye8, W_out[HALF:])
    bbig = jnp.tile(b_out, 8).reshape(1, 8 * NUM_CLASSES)
    pa = outa.reshape(NPAD // 8, 128)
    pb = outb.reshape(NPAD // 8, 128)
    packed = _head(pa, pb, wbig_a, wbig_b, bbig)
    return packed.reshape(N, NUM_CLASSES)


# speedup vs baseline: 12.2696x; 1.1668x over previous
import functools

import jax
import jax.numpy as jnp
from jax import lax
from jax.experimental import pallas as pl
from jax.experimental.pallas import tpu as pltpu
from jax.experimental.pallas import tpu_sc as plsc

N = 100000
E = 1600000
B = 128
HIDDEN = 32
HALF = 16
NUM_CLASSES = 2

ROWS = 512
GRID1 = 196
NPAD = GRID1 * ROWS

NS = 16
ROWS_PER_TILE = NPAD // NS
CHUNK = 128
IB = 4
NBLK = 196
CHUNKS_PER_TILE = IB * NBLK
EDGES_PER_TILE = CHUNKS_PER_TILE * CHUNK
EPAD = NS * EDGES_PER_TILE


def _enc_body(xb_ref, bt_ref, xg_ref, wl_ref, bl_ref, wg_ref, bg_ref,
              wm_ref, bm_ref, wmsg_ref, bmsg_ref, wself_ref, bself_ref,
              w_ref, m_ref):
    f32 = jnp.float32
    g = jnp.maximum(
        jnp.dot(xg_ref[...], wg_ref[...], preferred_element_type=f32) + bg_ref[...], 0.0)
    hl = jnp.maximum(
        jnp.dot(xb_ref[...], wl_ref[...], preferred_element_type=f32) + bl_ref[...], 0.0)
    bt = bt_ref[0, 0, :]
    oh = (bt[:, None] == lax.broadcasted_iota(jnp.int32, (ROWS, B), 1)).astype(f32)
    hg = jnp.dot(oh, g, preferred_element_type=f32)
    hcat = jnp.concatenate([hl, hg, hl * hg], axis=1)
    h0 = jnp.maximum(
        jnp.dot(hcat, wm_ref[...], preferred_element_type=f32) + bm_ref[...], 0.0)
    m0 = jnp.maximum(
        jnp.dot(h0, wmsg_ref[...], preferred_element_type=f32) + bmsg_ref[...], 0.0)
    t0 = m0 + jnp.dot(h0, wself_ref[...], preferred_element_type=f32) + bself_ref[...]
    w_ref[...] = jnp.concatenate(
        [t0, jnp.zeros((ROWS, 128 - HIDDEN), f32)], axis=1)
    m_ref[...] = jnp.concatenate(
        [m0, jnp.zeros((ROWS, 128 - HIDDEN), f32)], axis=1)


def _whole(shape):
    nd = len(shape)
    return pl.BlockSpec(shape, lambda i: (0,) * nd)


def _encoder(x_local, batch3, x_global, wl, bl, wg, bg, wm, bm, wmsg, bmsg,
             wself, bself):
    return pl.pallas_call(
        _enc_body,
        grid=(GRID1,),
        in_specs=[
            pl.BlockSpec((ROWS, 128), lambda i: (i, 0)),
            pl.BlockSpec((1, 1, ROWS), lambda i: (i, 0, 0)),
            _whole((B, 64)),
            _whole((128, HIDDEN)), _whole((1, HIDDEN)),
            _whole((64, HIDDEN)), _whole((1, HIDDEN)),
            _whole((3 * HIDDEN, HIDDEN)), _whole((1, HIDDEN)),
            _whole((HIDDEN, HIDDEN)), _whole((1, HIDDEN)),
            _whole((HIDDEN, HIDDEN)), _whole((1, HIDDEN)),
        ],
        out_specs=[pl.BlockSpec((ROWS, 128), lambda i: (i, 0))] * 2,
        out_shape=[jax.ShapeDtypeStruct((NPAD, 128), jnp.float32)] * 2,
    )(x_local, batch3, x_global, wl, bl, wg, bg, wm, bm, wmsg, bmsg, wself, bself)


def _sc_body(flat, wide, srca2d, srcb2d, dst2d, outa, outb,
             sidx, didx, rows, acc, semg0, semg1, sems0, sems1):
    c = lax.axis_index("c")
    s = lax.axis_index("s")
    semg = (semg0, semg1)
    sems = (sems0, sems1)

    def run(src2d, col, out):
        r0 = s * ROWS_PER_TILE
        pltpu.sync_copy(wide.at[pl.ds(r0, ROWS_PER_TILE), pl.ds(col, HALF)],
                        acc.at[pl.ds(r0, ROWS_PER_TILE)])
        plsc.subcore_barrier()
        row_base = s * CHUNKS_PER_TILE

        def load_idx(b, p):
            pltpu.sync_copy(src2d.at[pl.ds(row_base + b * IB, IB)], sidx.at[p])
            pltpu.sync_copy(dst2d.at[pl.ds(row_base + b * IB, IB)], didx.at[p])

        def fire_gathers(p):
            for jj in range(IB):
                pltpu.async_copy(
                    flat.at[sidx.at[p, jj]], rows.at[p, jj], semg[p])

        def drain_gathers(p):
            for jj in range(IB):
                pltpu.make_async_copy(
                    flat.at[sidx.at[p, jj]], rows.at[p, jj], semg[p]).wait()

        def fire_scatters(p):
            for jj in range(IB):
                pltpu.async_copy(rows.at[p, jj], acc.at[didx.at[p, jj]],
                                 sems[p], add=True)

        def drain_scatters(p):
            for jj in range(IB):
                pltpu.make_async_copy(
                    rows.at[p, jj], acc.at[didx.at[p, jj]], sems[p]).wait()

        load_idx(jnp.int32(0), 0)
        fire_gathers(0)

        def pair2(bb, carry):
            b0 = 2 * bb
            @pl.when(bb > 0)
            def _():
                drain_scatters(1)
            load_idx(b0 + 1, 1)
            fire_gathers(1)
            drain_gathers(0)
            fire_scatters(0)
            drain_scatters(0)
            load_idx(b0 + 2, 0)
            fire_gathers(0)
            drain_gathers(1)
            fire_scatters(1)
            return carry

        npair = NBLK // 2 - 1
        lax.fori_loop(0, npair, pair2, 0)
        drain_scatters(1)
        load_idx(jnp.int32(NBLK - 1), 1)
        fire_gathers(1)
        drain_gathers(0)
        fire_scatters(0)
        drain_scatters(0)
        drain_gathers(1)
        fire_scatters(1)
        drain_scatters(1)
        plsc.subcore_barrier()
        pltpu.sync_copy(acc.at[pl.ds(r0, ROWS_PER_TILE)],
                        out.at[pl.ds(r0, ROWS_PER_TILE)])

    @pl.when(c == 0)
    def _():
        run(srca2d, 0, outa)

    @pl.when(c == 1)
    def _():
        run(srcb2d, HALF, outb)


@functools.partial(
    pl.kernel,
    out_type=(jax.ShapeDtypeStruct((NPAD, HALF), jnp.float32),) * 2,
    mesh=plsc.VectorSubcoreMesh(core_axis_name="c", subcore_axis_name="s"),
    scratch_types=[
        pltpu.VMEM((2, IB, CHUNK), jnp.int32),
        pltpu.VMEM((2, IB, CHUNK), jnp.int32),
        pltpu.VMEM((2, IB, CHUNK, HALF), jnp.float32),
        pltpu.VMEM_SHARED((NPAD, HALF), jnp.float32),
        pltpu.SemaphoreType.DMA,
        pltpu.SemaphoreType.DMA,
        pltpu.SemaphoreType.DMA,
        pltpu.SemaphoreType.DMA,
    ],
    compiler_params=pltpu.CompilerParams(use_tc_tiling_on_sc=False),
)
def _sc_edge(flat, wide, srca2d, srcb2d, dst2d, outa, outb,
             sidx, didx, rows, acc, semg0, semg1, sems0, sems1):
    _sc_body(flat, wide, srca2d, srcb2d, dst2d, outa, outb,
             sidx, didx, rows, acc, semg0, semg1, sems0, sems1)


def _head_body(a_ref, b_ref, wa_ref, wb_ref, bo_ref, out_ref):
    ha = jnp.maximum(a_ref[...], 0.0)
    hb = jnp.maximum(b_ref[...], 0.0)
    out_ref[...] = (jnp.dot(ha, wa_ref[...], preferred_element_type=jnp.float32)
                    + jnp.dot(hb, wb_ref[...], preferred_element_type=jnp.float32)
                    + bo_ref[...])


def _head(pa, pb, wbig_a, wbig_b, bbig):
    return pl.pallas_call(
        _head_body,
        grid=(GRID1,),
        in_specs=[
            pl.BlockSpec((ROWS // 8, 128), lambda i: (i, 0)),
            pl.BlockSpec((ROWS // 8, 128), lambda i: (i, 0)),
            _whole((128, 8 * NUM_CLASSES)),
            _whole((128, 8 * NUM_CLASSES)),
            _whole((1, 8 * NUM_CLASSES)),
        ],
        out_specs=pl.BlockSpec((ROWS // 8, 8 * NUM_CLASSES), lambda i: (i, 0)),
        out_shape=jax.ShapeDtypeStruct((N * NUM_CLASSES // 16, 8 * NUM_CLASSES),
                                       jnp.float32),
    )(pa, pb, wbig_a, wbig_b, bbig)


@jax.jit
def kernel(x_local, x_global, edge_index, batch, W_local, b_local, W_global,
           b_global, W_mix, b_mix, W_msg, b_msg, W_self, b_self, W_out, b_out):
    batch3 = jnp.pad(batch, (0, NPAD - N)).reshape(GRID1, 1, ROWS)
    pad_e = jnp.full((EPAD - E,), N, jnp.int32)
    srca = jnp.concatenate([edge_index[0], pad_e]) * 8
    srcb = srca + 1
    srca2d = srca.reshape(EPAD // CHUNK, CHUNK)
    srcb2d = srcb.reshape(EPAD // CHUNK, CHUNK)
    dst2d = jnp.concatenate([edge_index[1], pad_e]).reshape(EPAD // CHUNK, CHUNK)
    row = lambda v: v.reshape(1, -1)

    wide, m0w = _encoder(
        x_local, batch3, x_global, W_local, row(b_local), W_global, row(b_global),
        W_mix, row(b_mix), W_msg, row(b_msg), W_self, row(b_self))
    flat = m0w.reshape(8 * NPAD, HALF)
    outa, outb = _sc_edge(flat, wide, srca2d, srcb2d, dst2d)

    eye8 = jnp.eye(8, dtype=jnp.float32)
    wbig_a = jnp.kron(eye8, W_out[:HALF])
    wbig_b = jnp.kron(eye8, W_out[HALF:])
    bbig = jnp.tile(b_out, 8).reshape(1, 8 * NUM_CLASSES)
    pa = outa.reshape(NPAD // 8, 128)
    pb = outb.reshape(NPAD // 8, 128)
    packed = _head(pa, pb, wbig_a, wbig_b, bbig)
    return packed.reshape(N, NUM_CLASSES)


# speedup vs baseline: 12.8115x; 1.0442x over previous
import functools

import jax
import jax.numpy as jnp
from jax import lax
from jax.experimental import pallas as pl
from jax.experimental.pallas import tpu as pltpu
from jax.experimental.pallas import tpu_sc as plsc

N = 100000
E = 1600000
B = 128
HIDDEN = 32
HALF = 16
NUM_CLASSES = 2

ROWS = 512
GRID1 = 196
NPAD = GRID1 * ROWS

NS = 16
ROWS_PER_TILE = NPAD // NS
CHUNK = 128
IB = 4
NBLK = 209
CHUNKS_PER_TILE = IB * NBLK
EDGES_PER_TILE = CHUNKS_PER_TILE * CHUNK
EPAD = NS * EDGES_PER_TILE


def _enc_body(xb_ref, bt_ref, xg_ref, wl_ref, bl_ref, wg_ref, bg_ref,
              wm_ref, bm_ref, wmsg_ref, bmsg_ref, wself_ref, bself_ref,
              m_ref):
    f32 = jnp.float32
    g = jnp.maximum(
        jnp.dot(xg_ref[...], wg_ref[...], preferred_element_type=f32) + bg_ref[...], 0.0)
    hl = jnp.maximum(
        jnp.dot(xb_ref[...], wl_ref[...], preferred_element_type=f32) + bl_ref[...], 0.0)
    bt = bt_ref[0, 0, :]
    oh = (bt[:, None] == lax.broadcasted_iota(jnp.int32, (ROWS, B), 1)).astype(f32)
    hg = jnp.dot(oh, g, preferred_element_type=f32)
    hcat = jnp.concatenate([hl, hg, hl * hg], axis=1)
    h0 = jnp.maximum(
        jnp.dot(hcat, wm_ref[...], preferred_element_type=f32) + bm_ref[...], 0.0)
    m0 = jnp.maximum(
        jnp.dot(h0, wmsg_ref[...], preferred_element_type=f32) + bmsg_ref[...], 0.0)
    t0 = m0 + jnp.dot(h0, wself_ref[...], preferred_element_type=f32) + bself_ref[...]
    m_ref[...] = jnp.concatenate(
        [m0, t0, jnp.zeros((ROWS, 128 - 2 * HIDDEN), f32)], axis=1)


def _whole(shape):
    nd = len(shape)
    return pl.BlockSpec(shape, lambda i: (0,) * nd)


def _encoder(x_local, batch3, x_global, wl, bl, wg, bg, wm, bm, wmsg, bmsg,
             wself, bself):
    return pl.pallas_call(
        _enc_body,
        grid=(GRID1,),
        in_specs=[
            pl.BlockSpec((ROWS, 128), lambda i: (i, 0)),
            pl.BlockSpec((1, 1, ROWS), lambda i: (i, 0, 0)),
            _whole((B, 64)),
            _whole((128, HIDDEN)), _whole((1, HIDDEN)),
            _whole((64, HIDDEN)), _whole((1, HIDDEN)),
            _whole((3 * HIDDEN, HIDDEN)), _whole((1, HIDDEN)),
            _whole((HIDDEN, HIDDEN)), _whole((1, HIDDEN)),
            _whole((HIDDEN, HIDDEN)), _whole((1, HIDDEN)),
        ],
        out_specs=pl.BlockSpec((ROWS, 128), lambda i: (i, 0)),
        out_shape=jax.ShapeDtypeStruct((NPAD, 128), jnp.float32),
    )(x_local, batch3, x_global, wl, bl, wg, bg, wm, bm, wmsg, bmsg, wself, bself)


def _sc_body(flat, zeros, srca2d, srcb2d, dst2d, outa, outb,
             sidx, didx, rows, acc, semg0, semg1, sems0, sems1):
    c = lax.axis_index("c")
    s = lax.axis_index("s")
    semg = (semg0, semg1)
    sems = (sems0, sems1)

    def run(src2d, out):
        r0 = s * ROWS_PER_TILE
        pltpu.sync_copy(zeros.at[pl.ds(r0, ROWS_PER_TILE)],
                        acc.at[pl.ds(r0, ROWS_PER_TILE)])
        plsc.subcore_barrier()
        row_base = s * CHUNKS_PER_TILE

        def load_idx(b, p):
            pltpu.sync_copy(src2d.at[pl.ds(row_base + b * IB, IB)], sidx.at[p])
            pltpu.sync_copy(dst2d.at[pl.ds(row_base + b * IB, IB)], didx.at[p])

        def fire_gathers(p):
            for jj in range(IB):
                pltpu.async_copy(
                    flat.at[sidx.at[p, jj]], rows.at[p, jj], semg[p])

        def drain_gathers(p):
            for jj in range(IB):
                pltpu.make_async_copy(
                    flat.at[sidx.at[p, jj]], rows.at[p, jj], semg[p]).wait()

        def fire_scatters(p):
            for jj in range(IB):
                pltpu.async_copy(rows.at[p, jj], acc.at[didx.at[p, jj]],
                                 sems[p], add=True)

        def drain_scatters(p):
            for jj in range(IB):
                pltpu.make_async_copy(
                    rows.at[p, jj], acc.at[didx.at[p, jj]], sems[p]).wait()

        load_idx(jnp.int32(0), 0)
        fire_gathers(0)

        def pair2(bb, carry):
            b0 = 2 * bb
            @pl.when(bb > 0)
            def _():
                drain_scatters(1)
            load_idx(b0 + 1, 1)
            fire_gathers(1)
            drain_gathers(0)
            fire_scatters(0)
            drain_scatters(0)
            load_idx(b0 + 2, 0)
            fire_gathers(0)
            drain_gathers(1)
            fire_scatters(1)
            return carry

        npair = (NBLK - 1) // 2
        lax.fori_loop(0, npair, pair2, 0)
        drain_scatters(1)
        drain_gathers(0)
        fire_scatters(0)
        drain_scatters(0)
        plsc.subcore_barrier()
        pltpu.sync_copy(acc.at[pl.ds(r0, ROWS_PER_TILE)],
                        out.at[pl.ds(r0, ROWS_PER_TILE)])

    @pl.when(c == 0)
    def _():
        run(srca2d, outa)

    @pl.when(c == 1)
    def _():
        run(srcb2d, outb)


@functools.partial(
    pl.kernel,
    out_type=(jax.ShapeDtypeStruct((NPAD, HALF), jnp.float32),) * 2,
    mesh=plsc.VectorSubcoreMesh(core_axis_name="c", subcore_axis_name="s"),
    scratch_types=[
        pltpu.VMEM((2, IB, CHUNK), jnp.int32),
        pltpu.VMEM((2, IB, CHUNK), jnp.int32),
        pltpu.VMEM((2, IB, CHUNK, HALF), jnp.float32),
        pltpu.VMEM_SHARED((NPAD, HALF), jnp.float32),
        pltpu.SemaphoreType.DMA,
        pltpu.SemaphoreType.DMA,
        pltpu.SemaphoreType.DMA,
        pltpu.SemaphoreType.DMA,
    ],
    compiler_params=pltpu.CompilerParams(use_tc_tiling_on_sc=False),
)
def _sc_edge(flat, zeros, srca2d, srcb2d, dst2d, outa, outb,
             sidx, didx, rows, acc, semg0, semg1, sems0, sems1):
    _sc_body(flat, zeros, srca2d, srcb2d, dst2d, outa, outb,
             sidx, didx, rows, acc, semg0, semg1, sems0, sems1)


def _head_body(a_ref, b_ref, wa_ref, wb_ref, bo_ref, out_ref):
    ha = jnp.maximum(a_ref[...], 0.0)
    hb = jnp.maximum(b_ref[...], 0.0)
    out_ref[...] = (jnp.dot(ha, wa_ref[...], preferred_element_type=jnp.float32)
                    + jnp.dot(hb, wb_ref[...], preferred_element_type=jnp.float32)
                    + bo_ref[...])


HROWS = 784


def _head(pa, pb, wbig_a, wbig_b, bbig):
    return pl.pallas_call(
        _head_body,
        grid=(NPAD // 8 // HROWS,),
        in_specs=[
            pl.BlockSpec((HROWS, 128), lambda i: (i, 0)),
            pl.BlockSpec((HROWS, 128), lambda i: (i, 0)),
            _whole((128, 8 * NUM_CLASSES)),
            _whole((128, 8 * NUM_CLASSES)),
            _whole((1, 8 * NUM_CLASSES)),
        ],
        out_specs=pl.BlockSpec((HROWS, 8 * NUM_CLASSES), lambda i: (i, 0)),
        out_shape=jax.ShapeDtypeStruct((N * NUM_CLASSES // 16, 8 * NUM_CLASSES),
                                       jnp.float32),
    )(pa, pb, wbig_a, wbig_b, bbig)


@jax.jit
def kernel(x_local, x_global, edge_index, batch, W_local, b_local, W_global,
           b_global, W_mix, b_mix, W_msg, b_msg, W_self, b_self, W_out, b_out):
    batch3 = jnp.pad(batch, (0, NPAD - N)).reshape(GRID1, 1, ROWS)
    npadearr = EPAD - E - NPAD
    pad_e = jnp.full((npadearr,), N, jnp.int32)
    nodes = jnp.arange(NPAD, dtype=jnp.int32)
    srca = jnp.concatenate([edge_index[0] * 8, nodes * 8 + 2, pad_e * 8])
    srca2d = srca.reshape(EPAD // CHUNK, CHUNK)
    srcb2d = (srca + 1).reshape(EPAD // CHUNK, CHUNK)
    dst2d = jnp.concatenate([edge_index[1], nodes, pad_e]).reshape(
        EPAD // CHUNK, CHUNK)
    zeros = jnp.zeros((NPAD, HALF), jnp.float32)
    row = lambda v: v.reshape(1, -1)

    m0w = _encoder(
        x_local, batch3, x_global, W_local, row(b_local), W_global, row(b_global),
        W_mix, row(b_mix), W_msg, row(b_msg), W_self, row(b_self))
    flat = m0w.reshape(8 * NPAD, HALF)
    outa, outb = _sc_edge(flat, zeros, srca2d, srcb2d, dst2d)

    eye8 = jnp.eye(8, dtype=jnp.float32)
    wbig_a = jnp.kron(eye8, W_out[:HALF])
    wbig_b = jnp.kron(eye8, W_out[HALF:])
    bbig = jnp.tile(b_out, 8).reshape(1, 8 * NUM_CLASSES)
    pa = outa.reshape(NPAD // 8, 128)
    pb = outb.reshape(NPAD // 8, 128)
    packed = _head(pa, pb, wbig_a, wbig_b, bbig)
    return packed.reshape(N, NUM_CLASSES)


# speedup vs baseline: 13.4722x; 1.0516x over previous
import functools

import jax
import jax.numpy as jnp
from jax import lax
from jax.experimental import pallas as pl
from jax.experimental.pallas import tpu as pltpu
from jax.experimental.pallas import tpu_sc as plsc

N = 100000
E = 1600000
B = 128
HIDDEN = 32
HALF = 16
NUM_CLASSES = 2

ROWS = 1024
GRID1 = 98
NPAD = GRID1 * ROWS

NS = 16
ROWS_PER_TILE = NPAD // NS
CHUNK = 128
IB = 5
NBLK = 168
CHUNKS_PER_TILE = IB * NBLK
EDGES_PER_TILE = CHUNKS_PER_TILE * CHUNK
EPAD = NS * EDGES_PER_TILE


def _enc_body(xb_ref, bt_ref, xg_ref, wl_ref, bl_ref, wg_ref, bg_ref,
              wm_ref, bm_ref, wmsg_ref, bmsg_ref, wself_ref, bself_ref,
              m_ref):
    f32 = jnp.float32
    g = jnp.maximum(
        jnp.dot(xg_ref[...], wg_ref[...], preferred_element_type=f32) + bg_ref[...], 0.0)
    hl = jnp.maximum(
        jnp.dot(xb_ref[...], wl_ref[...], preferred_element_type=f32) + bl_ref[...], 0.0)
    bt = bt_ref[0, 0, :]
    oh = (bt[:, None] == lax.broadcasted_iota(jnp.int32, (ROWS, B), 1)).astype(f32)
    hg = jnp.dot(oh, g, preferred_element_type=f32)
    hcat = jnp.concatenate([hl, hg, hl * hg], axis=1)
    h0 = jnp.maximum(
        jnp.dot(hcat, wm_ref[...], preferred_element_type=f32) + bm_ref[...], 0.0)
    m0 = jnp.maximum(
        jnp.dot(h0, wmsg_ref[...], preferred_element_type=f32) + bmsg_ref[...], 0.0)
    t0 = m0 + jnp.dot(h0, wself_ref[...], preferred_element_type=f32) + bself_ref[...]
    m_ref[...] = jnp.concatenate(
        [m0, t0, jnp.zeros((ROWS, 128 - 2 * HIDDEN), f32)], axis=1)


def _whole(shape):
    nd = len(shape)
    return pl.BlockSpec(shape, lambda i: (0,) * nd)


def _encoder(x_local, batch3, x_global, wl, bl, wg, bg, wm, bm, wmsg, bmsg,
             wself, bself):
    return pl.pallas_call(
        _enc_body,
        grid=(GRID1,),
        in_specs=[
            pl.BlockSpec((ROWS, 128), lambda i: (i, 0)),
            pl.BlockSpec((1, 1, ROWS), lambda i: (i, 0, 0)),
            _whole((B, 64)),
            _whole((128, HIDDEN)), _whole((1, HIDDEN)),
            _whole((64, HIDDEN)), _whole((1, HIDDEN)),
            _whole((3 * HIDDEN, HIDDEN)), _whole((1, HIDDEN)),
            _whole((HIDDEN, HIDDEN)), _whole((1, HIDDEN)),
            _whole((HIDDEN, HIDDEN)), _whole((1, HIDDEN)),
        ],
        out_specs=pl.BlockSpec((ROWS, 128), lambda i: (i, 0)),
        out_shape=jax.ShapeDtypeStruct((NPAD, 128), jnp.float32),
    )(x_local, batch3, x_global, wl, bl, wg, bg, wm, bm, wmsg, bmsg, wself, bself)


def _sc_body(flat, zeros, srca2d, srcb2d, dst2d, outa, outb,
             sidx, didx, rows, acc, semg0, semg1, sems0, sems1):
    c = lax.axis_index("c")
    s = lax.axis_index("s")
    semg = (semg0, semg1)
    sems = (sems0, sems1)

    def run(src2d, out):
        r0 = s * ROWS_PER_TILE
        pltpu.sync_copy(zeros.at[pl.ds(r0, ROWS_PER_TILE)],
                        acc.at[pl.ds(r0, ROWS_PER_TILE)])
        plsc.subcore_barrier()
        row_base = s * CHUNKS_PER_TILE

        def load_idx(b, p):
            pltpu.sync_copy(src2d.at[pl.ds(row_base + b * IB, IB)], sidx.at[p])
            pltpu.sync_copy(dst2d.at[pl.ds(row_base + b * IB, IB)], didx.at[p])

        def fire_gathers(p):
            for jj in range(IB):
                pltpu.async_copy(
                    flat.at[sidx.at[p, jj]], rows.at[p, jj], semg[p])

        def drain_gathers(p):
            for jj in range(IB):
                pltpu.make_async_copy(
                    flat.at[sidx.at[p, jj]], rows.at[p, jj], semg[p]).wait()

        def fire_scatters(p):
            for jj in range(IB):
                pltpu.async_copy(rows.at[p, jj], acc.at[didx.at[p, jj]],
                                 sems[p], add=True)

        def drain_scatters(p):
            for jj in range(IB):
                pltpu.make_async_copy(
                    rows.at[p, jj], acc.at[didx.at[p, jj]], sems[p]).wait()

        load_idx(jnp.int32(0), 0)
        fire_gathers(0)

        def pair2(bb, carry):
            b0 = 2 * bb
            @pl.when(bb > 0)
            def _():
                drain_scatters(1)
            load_idx(b0 + 1, 1)
            fire_gathers(1)
            drain_gathers(0)
            fire_scatters(0)
            drain_scatters(0)
            load_idx(b0 + 2, 0)
            fire_gathers(0)
            drain_gathers(1)
            fire_scatters(1)
            return carry

        npair = NBLK // 2 - 1
        lax.fori_loop(0, npair, pair2, 0)
        drain_scatters(1)
        load_idx(jnp.int32(NBLK - 1), 1)
        fire_gathers(1)
        drain_gathers(0)
        fire_scatters(0)
        drain_scatters(0)
        drain_gathers(1)
        fire_scatters(1)
        drain_scatters(1)
        plsc.subcore_barrier()
        pltpu.sync_copy(acc.at[pl.ds(r0, ROWS_PER_TILE)],
                        out.at[pl.ds(r0, ROWS_PER_TILE)])

    @pl.when(c == 0)
    def _():
        run(srca2d, outa)

    @pl.when(c == 1)
    def _():
        run(srcb2d, outb)


@functools.partial(
    pl.kernel,
    out_type=(jax.ShapeDtypeStruct((NPAD, HALF), jnp.float32),) * 2,
    mesh=plsc.VectorSubcoreMesh(core_axis_name="c", subcore_axis_name="s"),
    scratch_types=[
        pltpu.VMEM((2, IB, CHUNK), jnp.int32),
        pltpu.VMEM((2, IB, CHUNK), jnp.int32),
        pltpu.VMEM((2, IB, CHUNK, HALF), jnp.float32),
        pltpu.VMEM_SHARED((NPAD, HALF), jnp.float32),
        pltpu.SemaphoreType.DMA,
        pltpu.SemaphoreType.DMA,
        pltpu.SemaphoreType.DMA,
        pltpu.SemaphoreType.DMA,
    ],
    compiler_params=pltpu.CompilerParams(use_tc_tiling_on_sc=False),
)
def _sc_edge(flat, zeros, srca2d, srcb2d, dst2d, outa, outb,
             sidx, didx, rows, acc, semg0, semg1, sems0, sems1):
    _sc_body(flat, zeros, srca2d, srcb2d, dst2d, outa, outb,
             sidx, didx, rows, acc, semg0, semg1, sems0, sems1)


def _head_body(a_ref, b_ref, wa_ref, wb_ref, bo_ref, out_ref):
    ha = jnp.maximum(a_ref[...], 0.0)
    hb = jnp.maximum(b_ref[...], 0.0)
    out_ref[...] = (jnp.dot(ha, wa_ref[...], preferred_element_type=jnp.float32)
                    + jnp.dot(hb, wb_ref[...], preferred_element_type=jnp.float32)
                    + bo_ref[...])


HROWS = 784


def _head(pa, pb, wbig_a, wbig_b, bbig):
    return pl.pallas_call(
        _head_body,
        grid=(NPAD // 8 // HROWS,),
        in_specs=[
            pl.BlockSpec((HROWS, 128), lambda i: (i, 0)),
            pl.BlockSpec((HROWS, 128), lambda i: (i, 0)),
            _whole((128, 8 * NUM_CLASSES)),
            _whole((128, 8 * NUM_CLASSES)),
            _whole((1, 8 * NUM_CLASSES)),
        ],
        out_specs=pl.BlockSpec((HROWS, 8 * NUM_CLASSES), lambda i: (i, 0)),
        out_shape=jax.ShapeDtypeStruct((N * NUM_CLASSES // 16, 8 * NUM_CLASSES),
                                       jnp.float32),
    )(pa, pb, wbig_a, wbig_b, bbig)


@jax.jit
def kernel(x_local, x_global, edge_index, batch, W_local, b_local, W_global,
           b_global, W_mix, b_mix, W_msg, b_msg, W_self, b_self, W_out, b_out):
    batch3 = jnp.pad(batch, (0, NPAD - N)).reshape(GRID1, 1, ROWS)
    npadearr = EPAD - E - NPAD
    pad_e = jnp.full((npadearr,), N, jnp.int32)
    nodes = jnp.arange(NPAD, dtype=jnp.int32)
    srca = jnp.concatenate([edge_index[0] * 8, nodes * 8 + 2, pad_e * 8])
    srca2d = srca.reshape(EPAD // CHUNK, CHUNK)
    srcb2d = (srca + 1).reshape(EPAD // CHUNK, CHUNK)
    dst2d = jnp.concatenate([edge_index[1], nodes, pad_e]).reshape(
        EPAD // CHUNK, CHUNK)
    zeros = jnp.zeros((NPAD, HALF), jnp.float32)
    row = lambda v: v.reshape(1, -1)

    m0w = _encoder(
        x_local, batch3, x_global, W_local, row(b_local), W_global, row(b_global),
        W_mix, row(b_mix), W_msg, row(b_msg), W_self, row(b_self))
    flat = m0w.reshape(8 * NPAD, HALF)
    outa, outb = _sc_edge(flat, zeros, srca2d, srcb2d, dst2d)

    eye8 = jnp.eye(8, dtype=jnp.float32)
    wbig_a = jnp.kron(eye8, W_out[:HALF])
    wbig_b = jnp.kron(eye8, W_out[HALF:])
    bbig = jnp.tile(b_out, 8).reshape(1, 8 * NUM_CLASSES)
    pa = outa.reshape(NPAD // 8, 128)
    pb = outb.reshape(NPAD // 8, 128)
    packed = _head(pa, pb, wbig_a, wbig_b, bbig)
    return packed.reshape(N, NUM_CLASSES)
